# baseline (speedup 1.0000x reference)
import functools

import jax
import jax.numpy as jnp
from jax import lax
from jax.experimental import pallas as pl
from jax.experimental.pallas import tpu as pltpu
from jax.experimental.pallas import tpu_sc as plsc

_NUM_CORES = 2
_NUM_SUBCORES = 16
_NW = _NUM_CORES * _NUM_SUBCORES

_CHUNK_B = 8
_SUB = 104
_LANES = 16
_DIV_M = 40330


@functools.cache
def _build(bsz: int, fields: int, dim: int, vocab: int):
    wide = 2 * dim
    chunk = _CHUNK_B * fields
    nsub = chunk // _SUB
    strips = chunk // _LANES
    n_chunks = bsz // _CHUNK_B
    assert bsz % _CHUNK_B == 0 and n_chunks % _NW == 0
    assert chunk == nsub * _SUB and chunk % _LANES == 0
    nchunks_w = n_chunks // _NW
    rows_w = nchunks_w * chunk
    nvec = dim // _LANES

    scratch = [
        pltpu.VMEM((rows_w,), jnp.int32),
        pltpu.VMEM((rows_w,), jnp.int32),
        pltpu.VMEM((chunk, wide), jnp.float32),
        pltpu.VMEM((chunk, wide), jnp.float32),
        pltpu.VMEM((_CHUNK_B, fields, dim), jnp.float32),
        pltpu.SemaphoreType.DMA,
        pltpu.SemaphoreType.DMA,
    ]

    @functools.partial(
        pl.kernel,
        mesh=plsc.VectorSubcoreMesh(core_axis_name="c", subcore_axis_name="s"),
        out_type=jax.ShapeDtypeStruct((bsz, fields, dim), jnp.float32),
        scratch_types=scratch,
    )
    def emb(idx_hbm, table_hbm, out_hbm, cols_v, idx2, buf0, buf1,
            stage, gsem0, gsem1):
        bufs = (buf0, buf1)
        gsems = (gsem0, gsem1)
        wid = lax.axis_index("s") * _NUM_CORES + lax.axis_index("c")
        chunk0 = wid * nchunks_w

        pltpu.sync_copy(idx_hbm.at[pl.ds(wid * rows_w, rows_w)], cols_v)

        def split_strip(t, carry):
            v = cols_v[pl.ds(t * _LANES, _LANES)]
            idx2[pl.ds(t * _LANES, _LANES)] = lax.shift_right_logical(v, 1)
            cols_v[pl.ds(t * _LANES, _LANES)] = lax.shift_left(
                lax.bitwise_and(v, 1), 6)
            return carry
        lax.fori_loop(0, rows_w // _LANES, split_strip, 0)

        def start_gather(j, b):
            for k in range(nsub):
                pltpu.async_copy(
                    table_hbm.at[idx2.at[pl.ds(j * chunk + k * _SUB, _SUB)]],
                    bufs[b].at[pl.ds(k * _SUB, _SUB)],
                    gsems[b])

        def wait_gather(j, b):
            for k in range(nsub):
                pltpu.make_async_copy(
                    table_hbm.at[idx2.at[pl.ds(j * chunk + k * _SUB, _SUB)]],
                    bufs[b].at[pl.ds(k * _SUB, _SUB)],
                    gsems[b]).wait()

        start_gather(0, 0)
        start_gather(1, 1)

        def process(j, carry):
            for b in range(2):
                @pl.when(lax.rem(j, 2) == b)
                def _():
                    wait_gather(j, b)

                    def extract(s, carry2):
                        cvec = cols_v[pl.ds(j * chunk + s * _LANES, _LANES)]
                        for ell in range(_LANES):
                            r = s * _LANES + ell
                            c0 = cvec[ell]
                            k = lax.shift_right_logical(r * _DIV_M, 20)
                            f = r - k * fields
                            for u in range(nvec):
                                stage[k, f, pl.ds(u * _LANES, _LANES)] = (
                                    bufs[b][r, pl.ds(c0 + u * _LANES,
                                                     _LANES)])
                        return carry2
                    lax.fori_loop(0, strips, extract, 0)

                    nxt = j + 2

                    @pl.when(nxt < nchunks_w)
                    def _():
                        start_gather(nxt, b)

                    pltpu.sync_copy(
                        stage,
                        out_hbm.at[pl.ds((chunk0 + j) * _CHUNK_B, _CHUNK_B)])
            return carry

        lax.fori_loop(0, nchunks_w, process, 0)

    return emb


def kernel(token_ids, embedding):
    bsz, fields = token_ids.shape
    vocab, dim = embedding.shape
    emb = _build(bsz, fields, dim, vocab)
    idx = token_ids.reshape(bsz * fields).astype(jnp.int32)
    table2 = embedding.reshape(vocab // 2, 2 * dim)
    return emb(idx, table2)


# speedup vs baseline: 1.1154x; 1.1154x over previous
import functools

import jax
import jax.numpy as jnp
from jax import lax
from jax.experimental import pallas as pl
from jax.experimental.pallas import tpu as pltpu
from jax.experimental.pallas import tpu_sc as plsc

NBUF = 4

_NUM_CORES = 2
_NUM_SUBCORES = 16
_NW = _NUM_CORES * _NUM_SUBCORES


@functools.cache
def _build(bsz: int, fields: int, dim: int):
    n_rows = bsz * fields
    chunk = (128 // fields) * fields
    n_chunks = n_rows // chunk
    assert n_rows % chunk == 0 and n_chunks % _NW == 0
    nchunks_w = n_chunks // _NW
    assert nchunks_w % NBUF == 0

    scratch = [pltpu.VMEM((nchunks_w, chunk), jnp.int32)]
    scratch += [pltpu.VMEM((chunk, dim), jnp.float32) for _ in range(NBUF)]
    scratch += [pltpu.SemaphoreType.DMA for _ in range(NBUF)]

    @functools.partial(
        pl.kernel,
        mesh=plsc.VectorSubcoreMesh(core_axis_name="c", subcore_axis_name="s"),
        out_type=jax.ShapeDtypeStruct((n_chunks, chunk, dim), jnp.float32),
        scratch_types=scratch,
        compiler_params=pltpu.CompilerParams(use_tc_tiling_on_sc=False),
    )
    def emb(idx_hbm, table_hbm, out_hbm, idx_v, *rest):
        bufs = rest[:NBUF]
        sems = rest[NBUF:]
        wid = lax.axis_index("s") * _NUM_CORES + lax.axis_index("c")
        chunk0 = wid * nchunks_w

        pltpu.sync_copy(idx_hbm.at[pl.ds(chunk0, nchunks_w)], idx_v)

        for b in range(NBUF):
            pltpu.async_copy(table_hbm.at[idx_v.at[b]], bufs[b], sems[b])

        def round_(i, carry):
            for b in range(NBUF):
                j = i * NBUF + b
                pltpu.make_async_copy(
                    table_hbm.at[idx_v.at[j]], bufs[b], sems[b]).wait()
                pltpu.sync_copy(bufs[b], out_hbm.at[chunk0 + j])
                nxt = j + NBUF

                @pl.when(nxt < nchunks_w)
                def _():
                    pltpu.async_copy(
                        table_hbm.at[idx_v.at[nxt]], bufs[b], sems[b])
            return carry

        lax.fori_loop(0, nchunks_w // NBUF, round_, 0)

    return emb, chunk, n_chunks


def kernel(token_ids, embedding):
    bsz, fields = token_ids.shape
    _, dim = embedding.shape
    emb, chunk, n_chunks = _build(bsz, fields, dim)
    idx = token_ids.reshape(n_chunks, chunk).astype(jnp.int32)
    out = emb(idx, embedding)
    return out.reshape(bsz, fields, dim)
